# baseline (speedup 1.0000x reference)
import functools

import jax
import jax.numpy as jnp
from jax import lax
from jax.experimental import pallas as pl
from jax.experimental.pallas import tpu as pltpu
from jax.experimental.pallas import tpu_sc as plsc

_N_TOKENS = 32768
_DIM = 768
_CH = 64


def kernel(x, W, b):
    del W, b
    info = plsc.get_sparse_core_info()
    nc, ns = info.num_cores, info.num_subcores
    nw = nc * ns
    rows_per_w = _N_TOKENS // nw
    n_chunks = rows_per_w // _CH
    mesh = plsc.VectorSubcoreMesh(core_axis_name="c", subcore_axis_name="s")

    @functools.partial(
        pl.kernel, mesh=mesh,
        out_type=jax.ShapeDtypeStruct((_N_TOKENS, _DIM), jnp.float32),
        scratch_types=[
            pltpu.VMEM((2, _CH, _DIM), jnp.float32),
            pltpu.SemaphoreType.DMA((2,)),
            pltpu.SemaphoreType.DMA((2,)),
        ],
    )
    def sc_copy(x_hbm, o_hbm, buf, insem, outsem):
        wid = lax.axis_index("s") * nc + lax.axis_index("c")
        base = wid * rows_per_w

        def in_dma(chunk, slot):
            return pltpu.make_async_copy(
                x_hbm.at[pl.ds(base + chunk * _CH, _CH), :], buf.at[slot],
                insem.at[slot])

        def out_dma(chunk, slot):
            return pltpu.make_async_copy(
                buf.at[slot], o_hbm.at[pl.ds(base + chunk * _CH, _CH), :],
                outsem.at[slot])

        in_dma(0, 0).start()
        in_dma(1, 1).start()
        for i in range(n_chunks):
            slot = i % 2
            in_dma(i, slot).wait()
            out_dma(i, slot).start()
            if i + 2 < n_chunks:
                out_dma(i, slot).wait()
                in_dma(i + 2, slot).start()
        for i in (n_chunks - 2, n_chunks - 1):
            out_dma(i, i % 2).wait()

    return sc_copy(x)
